# baseline (speedup 1.0000x reference)
import jax
import jax.numpy as jnp
from jax import lax
from jax.experimental import pallas as pl
from jax.experimental.pallas import tpu as pltpu
from jax.experimental.pallas import tpu_sc as plsc

NUM_ROWS = 1_000_000
DIM = 32
LANES = 128

_info = plsc.get_sparse_core_info()
_NC = _info.num_cores
_NS = _info.num_subcores
_NW = _NC * _NS

_JB = 4
_IBLK = 4
_GDEPTH = 4


def _gather_body(table_hbm, xt_hbm, out_hbm, idx_v, rows_v, outt_v, *sems):
    gsems, wsems = sems[:_GDEPTH], sems[_GDEPTH:]
    b1, b0 = xt_hbm.shape
    slab = _IBLK * LANES
    n_jc = b1 // _JB
    kt_n = DIM // 8
    it_n = b0 // LANES
    wid = lax.axis_index("s") * _NC + lax.axis_index("c")
    i0 = wid * slab

    kv_lo = lax.iota(jnp.int32, 16)
    kv_hi = lax.iota(jnp.int32, 16) + 16

    _UNR = 8

    def transpose_unit(gbuf, obuf):
        def istep(ii, carry):
            i = ii * _UNR
            for d in range(_UNR):
                ivec = jnp.zeros((16,), jnp.int32) + (i + d)
                v0 = rows_v[gbuf, i + d, pl.ds(0, 16)]
                v1 = rows_v[gbuf, i + d, pl.ds(16, 16)]
                plsc.store_scatter(outt_v.at[obuf], [kv_lo, ivec], v0)
                plsc.store_scatter(outt_v.at[obuf], [kv_hi, ivec], v1)
            return carry
        lax.fori_loop(0, LANES // _UNR, istep, 0)

    def start_gather(jj, b, buf):
        return pltpu.async_copy(
            table_hbm.at[idx_v.at[jj, pl.ds(LANES * b, LANES)]],
            rows_v.at[buf],
            gsems[buf],
        )

    def start_writeback(j, b, buf):
        cps = []
        for kt in range(kt_n):
            r0 = (j * kt_n + kt) * it_n * 8 + (wid * _IBLK + b) * 8
            cps.append(pltpu.async_copy(
                outt_v.at[buf, pl.ds(8 * kt, 8), pl.ds(0, LANES)],
                out_hbm.at[pl.ds(r0, 8)],
                wsems[buf],
            ))
        return cps

    def jchunk(jc, carry):
        pltpu.sync_copy(xt_hbm.at[pl.ds(jc * _JB, _JB), pl.ds(i0, slab)],
                        idx_v)
        n_u = _JB * _IBLK
        gcps = [None] * n_u
        wcps = [None] * n_u
        for p in range(_GDEPTH - 1):
            jj, b = divmod(p, _IBLK)
            gcps[p] = start_gather(jj, b, p % _GDEPTH)
        for u in range(n_u):
            if u + _GDEPTH - 1 < n_u:
                jj, b = divmod(u + _GDEPTH - 1, _IBLK)
                gcps[u + _GDEPTH - 1] = start_gather(
                    jj, b, (u + _GDEPTH - 1) % _GDEPTH)
            gcps[u].wait()
            if u >= 2:
                for cp in wcps[u - 2]:
                    cp.wait()
            transpose_unit(u % _GDEPTH, u % 2)
            jj, b = divmod(u, _IBLK)
            wcps[u] = start_writeback(jc * _JB + jj, b, u % 2)
        for cp in wcps[n_u - 2] + wcps[n_u - 1]:
            cp.wait()
        return carry

    lax.fori_loop(0, n_jc, jchunk, 0)


def kernel(x, weight):
    b0, b1 = x.shape
    xt = x.T.astype(jnp.int32)
    gather = pl.kernel(
        _gather_body,
        out_type=jax.ShapeDtypeStruct((b0 * b1 * DIM // LANES, LANES),
                                      jnp.float32),
        mesh=plsc.VectorSubcoreMesh(core_axis_name="c", subcore_axis_name="s"),
        scratch_types=[
            pltpu.VMEM((_JB, _IBLK * LANES), jnp.int32),
            pltpu.VMEM((_GDEPTH, LANES, DIM), jnp.float32),
            pltpu.VMEM((2, DIM, LANES + 1), jnp.float32),
        ] + [pltpu.SemaphoreType.DMA] * (_GDEPTH + 2),
        compiler_params=pltpu.CompilerParams(use_tc_tiling_on_sc=False,
                                             needs_layout_passes=False),
    )
    out2d = gather(weight, xt)
    v5 = out2d.reshape(b1, DIM // 8, b0 // LANES, 8, LANES)
    return v5.transpose(2, 4, 0, 1, 3).reshape(b0, b1, DIM)


# speedup vs baseline: 1.0249x; 1.0249x over previous
import jax
import jax.numpy as jnp
from jax import lax
from jax.experimental import pallas as pl
from jax.experimental.pallas import tpu as pltpu
from jax.experimental.pallas import tpu_sc as plsc

NUM_ROWS = 1_000_000
DIM = 32
LANES = 128

_info = plsc.get_sparse_core_info()
_NC = _info.num_cores
_NS = _info.num_subcores
_NW = _NC * _NS

_JB = 4
_IBLK = 4
_GDEPTH = 8


def _gather_body(table_hbm, xt_hbm, out_hbm, idx_v, rows_v, outt_v, *sems):
    gsems, wsems = sems[:_GDEPTH], sems[_GDEPTH:]
    b1, b0 = xt_hbm.shape
    slab = _IBLK * LANES
    n_jc = b1 // _JB
    kt_n = DIM // 8
    it_n = b0 // LANES
    wid = lax.axis_index("s") * _NC + lax.axis_index("c")
    i0 = wid * slab

    kv_lo = lax.iota(jnp.int32, 16)
    kv_hi = lax.iota(jnp.int32, 16) + 16

    def transpose_unit(gbuf, obuf):
        def istep(i, carry):
            ivec = jnp.zeros((16,), jnp.int32) + i
            v0 = rows_v[gbuf, i, pl.ds(0, 16)]
            v1 = rows_v[gbuf, i, pl.ds(16, 16)]
            plsc.store_scatter(outt_v.at[obuf], [kv_lo, ivec], v0)
            plsc.store_scatter(outt_v.at[obuf], [kv_hi, ivec], v1)
            return carry
        lax.fori_loop(0, LANES, istep, 0)

    def start_gather(jj, b, buf):
        return pltpu.async_copy(
            table_hbm.at[idx_v.at[jj, pl.ds(LANES * b, LANES)]],
            rows_v.at[buf],
            gsems[buf],
        )

    def start_writeback(j, b, buf):
        cps = []
        for kt in range(kt_n):
            r0 = (j * kt_n + kt) * it_n * 8 + (wid * _IBLK + b) * 8
            cps.append(pltpu.async_copy(
                outt_v.at[buf, pl.ds(8 * kt, 8), pl.ds(0, LANES)],
                out_hbm.at[pl.ds(r0, 8)],
                wsems[buf],
            ))
        return cps

    def jchunk(jc, carry):
        pltpu.sync_copy(xt_hbm.at[pl.ds(jc * _JB, _JB), pl.ds(i0, slab)],
                        idx_v)
        n_u = _JB * _IBLK
        gcps = [None] * n_u
        wcps = [None] * n_u
        for p in range(_GDEPTH - 1):
            jj, b = divmod(p, _IBLK)
            gcps[p] = start_gather(jj, b, p % _GDEPTH)
        for u in range(n_u):
            if u + _GDEPTH - 1 < n_u:
                jj, b = divmod(u + _GDEPTH - 1, _IBLK)
                gcps[u + _GDEPTH - 1] = start_gather(
                    jj, b, (u + _GDEPTH - 1) % _GDEPTH)
            gcps[u].wait()
            if u >= 2:
                for cp in wcps[u - 2]:
                    cp.wait()
            transpose_unit(u % _GDEPTH, u % 2)
            jj, b = divmod(u, _IBLK)
            wcps[u] = start_writeback(jc * _JB + jj, b, u % 2)
        for cp in wcps[n_u - 2] + wcps[n_u - 1]:
            cp.wait()
        return carry

    lax.fori_loop(0, n_jc, jchunk, 0)


def kernel(x, weight):
    b0, b1 = x.shape
    xt = x.T.astype(jnp.int32)
    gather = pl.kernel(
        _gather_body,
        out_type=jax.ShapeDtypeStruct((b0 * b1 * DIM // LANES, LANES),
                                      jnp.float32),
        mesh=plsc.VectorSubcoreMesh(core_axis_name="c", subcore_axis_name="s"),
        scratch_types=[
            pltpu.VMEM((_JB, _IBLK * LANES), jnp.int32),
            pltpu.VMEM((_GDEPTH, LANES, DIM), jnp.float32),
            pltpu.VMEM((2, DIM, LANES + 1), jnp.float32),
        ] + [pltpu.SemaphoreType.DMA] * (_GDEPTH + 2),
        compiler_params=pltpu.CompilerParams(use_tc_tiling_on_sc=False,
                                             needs_layout_passes=False),
    )
    out2d = gather(weight, xt)
    v5 = out2d.reshape(b1, DIM // 8, b0 // LANES, 8, LANES)
    return v5.transpose(2, 4, 0, 1, 3).reshape(b0, b1, DIM)


# speedup vs baseline: 1.0761x; 1.0499x over previous
import jax
import jax.numpy as jnp
from jax import lax
from jax.experimental import pallas as pl
from jax.experimental.pallas import tpu as pltpu
from jax.experimental.pallas import tpu_sc as plsc

NUM_ROWS = 1_000_000
DIM = 32
LANES = 128

_info = plsc.get_sparse_core_info()
_NC = _info.num_cores
_NS = _info.num_subcores
_NW = _NC * _NS

_JB = 10
_IBLK = 4
_GDEPTH = 4


def _gather_body(table_hbm, xt_hbm, out_hbm, idx_v, rows_v, outt_v, *sems):
    gsems, wsems = sems[:_GDEPTH], sems[_GDEPTH:]
    b1, b0 = xt_hbm.shape
    slab = _IBLK * LANES
    n_jc = b1 // _JB
    kt_n = DIM // 8
    it_n = b0 // LANES
    wid = lax.axis_index("s") * _NC + lax.axis_index("c")
    i0 = wid * slab

    kv_lo = lax.iota(jnp.int32, 16)
    kv_hi = lax.iota(jnp.int32, 16) + 16

    def transpose_unit(gbuf, obuf):
        def istep(i, carry):
            ivec = jnp.zeros((16,), jnp.int32) + i
            v0 = rows_v[gbuf, i, pl.ds(0, 16)]
            v1 = rows_v[gbuf, i, pl.ds(16, 16)]
            plsc.store_scatter(outt_v.at[obuf], [kv_lo, ivec], v0)
            plsc.store_scatter(outt_v.at[obuf], [kv_hi, ivec], v1)
            return carry
        lax.fori_loop(0, LANES, istep, 0)

    def start_gather(jj, b, buf):
        return pltpu.async_copy(
            table_hbm.at[idx_v.at[jj, pl.ds(LANES * b, LANES)]],
            rows_v.at[buf],
            gsems[buf],
        )

    def start_writeback(j, b, buf):
        cps = []
        for kt in range(kt_n):
            r0 = (j * kt_n + kt) * it_n * 8 + (wid * _IBLK + b) * 8
            cps.append(pltpu.async_copy(
                outt_v.at[buf, pl.ds(8 * kt, 8), pl.ds(0, LANES)],
                out_hbm.at[pl.ds(r0, 8)],
                wsems[buf],
            ))
        return cps

    def jchunk(jc, carry):
        pltpu.sync_copy(xt_hbm.at[pl.ds(jc * _JB, _JB), pl.ds(i0, slab)],
                        idx_v)
        n_u = _JB * _IBLK
        gcps = [None] * n_u
        wcps = [None] * n_u
        for p in range(_GDEPTH - 1):
            jj, b = divmod(p, _IBLK)
            gcps[p] = start_gather(jj, b, p % _GDEPTH)
        for u in range(n_u):
            if u + _GDEPTH - 1 < n_u:
                jj, b = divmod(u + _GDEPTH - 1, _IBLK)
                gcps[u + _GDEPTH - 1] = start_gather(
                    jj, b, (u + _GDEPTH - 1) % _GDEPTH)
            gcps[u].wait()
            if u >= 2:
                for cp in wcps[u - 2]:
                    cp.wait()
            transpose_unit(u % _GDEPTH, u % 2)
            jj, b = divmod(u, _IBLK)
            wcps[u] = start_writeback(jc * _JB + jj, b, u % 2)
        for cp in wcps[n_u - 2] + wcps[n_u - 1]:
            cp.wait()
        return carry

    lax.fori_loop(0, n_jc, jchunk, 0)


def kernel(x, weight):
    b0, b1 = x.shape
    xt = x.T.astype(jnp.int32)
    gather = pl.kernel(
        _gather_body,
        out_type=jax.ShapeDtypeStruct((b0 * b1 * DIM // LANES, LANES),
                                      jnp.float32),
        mesh=plsc.VectorSubcoreMesh(core_axis_name="c", subcore_axis_name="s"),
        scratch_types=[
            pltpu.VMEM((_JB, _IBLK * LANES), jnp.int32),
            pltpu.VMEM((_GDEPTH, LANES, DIM), jnp.float32),
            pltpu.VMEM((2, DIM, LANES + 1), jnp.float32),
        ] + [pltpu.SemaphoreType.DMA] * (_GDEPTH + 2),
        compiler_params=pltpu.CompilerParams(use_tc_tiling_on_sc=False,
                                             needs_layout_passes=False),
    )
    out2d = gather(weight, xt)
    v5 = out2d.reshape(b1, DIM // 8, b0 // LANES, 8, LANES)
    return v5.transpose(2, 4, 0, 1, 3).reshape(b0, b1, DIM)


# speedup vs baseline: 1.0848x; 1.0081x over previous
import jax
import jax.numpy as jnp
from jax import lax
from jax.experimental import pallas as pl
from jax.experimental.pallas import tpu as pltpu
from jax.experimental.pallas import tpu_sc as plsc

NUM_ROWS = 1_000_000
DIM = 32
LANES = 128

_info = plsc.get_sparse_core_info()
_NC = _info.num_cores
_NS = _info.num_subcores
_NW = _NC * _NS

_JB = 20
_IBLK = 4
_GDEPTH = 4


def _gather_body(table_hbm, xt_hbm, out_hbm, idx_v, rows_v, outt_v, *sems):
    gsems, wsems = sems[:_GDEPTH], sems[_GDEPTH:]
    b1, b0 = xt_hbm.shape
    slab = _IBLK * LANES
    n_jc = b1 // _JB
    kt_n = DIM // 8
    it_n = b0 // LANES
    wid = lax.axis_index("s") * _NC + lax.axis_index("c")
    i0 = wid * slab

    kv_lo = lax.iota(jnp.int32, 16)
    kv_hi = lax.iota(jnp.int32, 16) + 16

    def transpose_unit(gbuf, obuf):
        def istep(i, carry):
            ivec = jnp.zeros((16,), jnp.int32) + i
            v0 = rows_v[gbuf, i, pl.ds(0, 16)]
            v1 = rows_v[gbuf, i, pl.ds(16, 16)]
            plsc.store_scatter(outt_v.at[obuf], [kv_lo, ivec], v0)
            plsc.store_scatter(outt_v.at[obuf], [kv_hi, ivec], v1)
            return carry
        lax.fori_loop(0, LANES, istep, 0)

    def start_gather(jj, b, buf):
        return pltpu.async_copy(
            table_hbm.at[idx_v.at[jj, pl.ds(LANES * b, LANES)]],
            rows_v.at[buf],
            gsems[buf],
        )

    def start_writeback(j, b, buf):
        cps = []
        for kt in range(kt_n):
            r0 = (j * kt_n + kt) * it_n * 8 + (wid * _IBLK + b) * 8
            cps.append(pltpu.async_copy(
                outt_v.at[buf, pl.ds(8 * kt, 8), pl.ds(0, LANES)],
                out_hbm.at[pl.ds(r0, 8)],
                wsems[buf],
            ))
        return cps

    def jchunk(jc, carry):
        pltpu.sync_copy(xt_hbm.at[pl.ds(jc * _JB, _JB), pl.ds(i0, slab)],
                        idx_v)
        n_u = _JB * _IBLK
        gcps = [None] * n_u
        wcps = [None] * n_u
        for p in range(_GDEPTH - 1):
            jj, b = divmod(p, _IBLK)
            gcps[p] = start_gather(jj, b, p % _GDEPTH)
        for u in range(n_u):
            if u + _GDEPTH - 1 < n_u:
                jj, b = divmod(u + _GDEPTH - 1, _IBLK)
                gcps[u + _GDEPTH - 1] = start_gather(
                    jj, b, (u + _GDEPTH - 1) % _GDEPTH)
            gcps[u].wait()
            if u >= 2:
                for cp in wcps[u - 2]:
                    cp.wait()
            transpose_unit(u % _GDEPTH, u % 2)
            jj, b = divmod(u, _IBLK)
            wcps[u] = start_writeback(jc * _JB + jj, b, u % 2)
        for cp in wcps[n_u - 2] + wcps[n_u - 1]:
            cp.wait()
        return carry

    lax.fori_loop(0, n_jc, jchunk, 0)


def kernel(x, weight):
    b0, b1 = x.shape
    xt = x.T.astype(jnp.int32)
    gather = pl.kernel(
        _gather_body,
        out_type=jax.ShapeDtypeStruct((b0 * b1 * DIM // LANES, LANES),
                                      jnp.float32),
        mesh=plsc.VectorSubcoreMesh(core_axis_name="c", subcore_axis_name="s"),
        scratch_types=[
            pltpu.VMEM((_JB, _IBLK * LANES), jnp.int32),
            pltpu.VMEM((_GDEPTH, LANES, DIM), jnp.float32),
            pltpu.VMEM((2, DIM, LANES + 1), jnp.float32),
        ] + [pltpu.SemaphoreType.DMA] * (_GDEPTH + 2),
        compiler_params=pltpu.CompilerParams(use_tc_tiling_on_sc=False,
                                             needs_layout_passes=False),
    )
    out2d = gather(weight, xt)
    v5 = out2d.reshape(b1, DIM // 8, b0 // LANES, 8, LANES)
    return v5.transpose(2, 4, 0, 1, 3).reshape(b0, b1, DIM)
